# Initial kernel scaffold; baseline (speedup 1.0000x reference)
import jax
import jax.numpy as jnp
from jax.experimental import pallas as pl


def kernel(x, edge_index, batch, W1, b1, W2, b2, W3, b3, Wl1, bl1, Wl2, bl2):
    raise NotImplementedError("write your pallas kernel here")


# speedup vs baseline: 9.3113x; 9.3113x over previous
import functools

import jax
import jax.numpy as jnp
from jax import lax
from jax.experimental import pallas as pl
from jax.experimental.pallas import tpu as pltpu
from jax.experimental.pallas import tpu_sc as plsc

N = 10000
E = 320000
G = 64
IN_CH = 128
HID = 64
HF = 128

NC = 2
NS = 16
NW = NC * NS
CH = 128

NP = 10240
RPT = NP // NS
EP = 323584
EPW = EP // NW
NCH = EPW // CH

_mesh = plsc.VectorSubcoreMesh(
    core_axis_name="c", subcore_axis_name="s", num_cores=NC, num_subcores=NS)


@functools.partial(
    pl.kernel,
    out_type=jax.ShapeDtypeStruct((NC, NP), jnp.float32),
    mesh=_mesh,
    scratch_types=[
        pltpu.VMEM_SHARED((NP,), jnp.float32),
        pltpu.VMEM((CH,), jnp.int32),
        pltpu.VMEM((CH,), jnp.float32),
    ],
)
def _deg(dst_hbm, ones_hbm, zn_hbm, out_hbm, acc, idx, ones_v):
    cid = lax.axis_index("c")
    sid = lax.axis_index("s")
    wid = sid * NC + cid
    pltpu.sync_copy(zn_hbm.at[pl.ds(sid * RPT, RPT)],
                    acc.at[pl.ds(sid * RPT, RPT)])
    pltpu.sync_copy(ones_hbm, ones_v)
    plsc.subcore_barrier()
    base = wid * EPW

    def body(c, carry):
        pltpu.sync_copy(dst_hbm.at[pl.ds(base + c * CH, CH)], idx)
        pltpu.sync_copy(ones_v, acc.at[idx], add=True)
        return carry

    lax.fori_loop(0, NCH, body, 0)
    plsc.subcore_barrier()
    pltpu.sync_copy(acc.at[pl.ds(sid * RPT, RPT)],
                    out_hbm.at[cid].at[pl.ds(sid * RPT, RPT)])


@functools.partial(
    pl.kernel,
    out_type=jax.ShapeDtypeStruct((NC, NP, HF), jnp.float32),
    mesh=_mesh,
    scratch_types=[
        pltpu.VMEM_SHARED((NP, HF), jnp.float32),
        pltpu.VMEM((CH,), jnp.int32),
        pltpu.VMEM((CH,), jnp.int32),
        pltpu.VMEM((CH, HF), jnp.float32),
        pltpu.SemaphoreType.DMA,
    ],
)
def _agg(hs_hbm, src_hbm, dst_hbm, zr_hbm, out_hbm, acc, isrc, idst, rows, sem):
    cid = lax.axis_index("c")
    sid = lax.axis_index("s")
    wid = sid * NC + cid
    pltpu.sync_copy(zr_hbm.at[pl.ds(sid * RPT, RPT)],
                    acc.at[pl.ds(sid * RPT, RPT)])
    plsc.subcore_barrier()
    base = wid * EPW

    def body(c, carry):
        e0 = base + c * CH
        pltpu.sync_copy(src_hbm.at[pl.ds(e0, CH)], isrc)
        pltpu.sync_copy(dst_hbm.at[pl.ds(e0, CH)], idst)
        pltpu.async_copy(hs_hbm.at[isrc], rows, sem).wait()
        pltpu.sync_copy(rows, acc.at[idst], add=True)
        return carry

    lax.fori_loop(0, NCH, body, 0)
    plsc.subcore_barrier()
    pltpu.sync_copy(acc.at[pl.ds(sid * RPT, RPT)],
                    out_hbm.at[cid].at[pl.ds(sid * RPT, RPT)])


def _dinv_of(pT_ref):
    return lax.rsqrt(1.0 + pT_ref[:, 0:1] + pT_ref[:, 1:2])


def _tc_first_body(pT_ref, xp_ref, w1_ref, out_ref):
    h = jnp.dot(xp_ref[...], w1_ref[...], preferred_element_type=jnp.float32)
    out_ref[...] = _dinv_of(pT_ref) * h


def _tc_mid_body(pT_ref, q_ref, hs_ref, b_ref, w_ref, out_ref):
    dinv = _dinv_of(pT_ref)
    agg = q_ref[0] + q_ref[1] + hs_ref[...]
    h = jnp.maximum(dinv * agg + b_ref[...], 0.0)
    out_ref[...] = dinv * jnp.dot(h, w_ref[...],
                                  preferred_element_type=jnp.float32)


def _tc_final_body(pT_ref, q_ref, hs_ref, b3_ref, bp_ref,
                   wl1_ref, bl1_ref, wl2_ref, bl2_ref, out_ref):
    dinv = _dinv_of(pT_ref)
    agg = q_ref[0] + q_ref[1] + hs_ref[...]
    h3 = jnp.maximum(dinv * agg + b3_ref[...], 0.0)
    gids = lax.broadcasted_iota(jnp.int32, (G, NP), 0)
    m = (bp_ref[...] == gids).astype(jnp.float32)
    counts = jnp.sum(m, axis=1, keepdims=True)
    sums = jnp.dot(m, h3, preferred_element_type=jnp.float32)
    g = sums / jnp.maximum(counts, 1.0)
    r = jnp.maximum(
        jnp.dot(g, wl1_ref[...], preferred_element_type=jnp.float32)
        + bl1_ref[...], 0.0)
    o = jnp.dot(r, wl2_ref[...], preferred_element_type=jnp.float32) \
        + bl2_ref[...]
    out_ref[...] = jax.nn.sigmoid(o)


_tc_first = pl.pallas_call(
    _tc_first_body, out_shape=jax.ShapeDtypeStruct((NP, HF), jnp.float32))
_tc_mid = pl.pallas_call(
    _tc_mid_body, out_shape=jax.ShapeDtypeStruct((NP, HF), jnp.float32))
_tc_final = pl.pallas_call(
    _tc_final_body, out_shape=jax.ShapeDtypeStruct((G, 1), jnp.float32))


def _padw(W):
    return jnp.zeros((HF, HF), jnp.float32).at[:W.shape[0], :W.shape[1]].set(W)


def _padb(b):
    return jnp.zeros((1, HF), jnp.float32).at[0, :b.shape[0]].set(b)


def kernel(x, edge_index, batch, W1, b1, W2, b2, W3, b3, Wl1, bl1, Wl2, bl2):
    f32 = jnp.float32
    src = edge_index[0].astype(jnp.int32)
    dst = edge_index[1].astype(jnp.int32)
    pad = jnp.full((EP - E,), N, jnp.int32)
    srcp = jnp.concatenate([src, pad])
    dstp = jnp.concatenate([dst, pad])
    xp = jnp.zeros((NP, IN_CH), f32).at[:N].set(x.astype(f32))
    zr = jnp.zeros((NP, HF), f32)
    zn = jnp.zeros((NP,), f32)
    ones = jnp.ones((CH,), f32)
    bp = jnp.concatenate(
        [batch.astype(jnp.int32), jnp.full((NP - N,), G, jnp.int32)]
    ).reshape(1, NP)

    p = _deg(dstp, ones, zn)
    pT = p.T
    W1p = jnp.zeros((IN_CH, HF), f32).at[:, :HID].set(W1)
    hs1 = _tc_first(pT, xp, W1p)
    q1 = _agg(hs1, srcp, dstp, zr)
    hs2 = _tc_mid(pT, q1, hs1, _padb(b1), _padw(W2))
    q2 = _agg(hs2, srcp, dstp, zr)
    hs3 = _tc_mid(pT, q2, hs2, _padb(b2), _padw(W3))
    q3 = _agg(hs3, srcp, dstp, zr)
    Wl1p = jnp.zeros((HF, HID // 2), f32).at[:HID].set(Wl1)
    return _tc_final(pT, q3, hs3, _padb(b3), bp,
                     Wl1p, bl1.reshape(1, HID // 2), Wl2, bl2.reshape(1, 1))
